# baseline (speedup 1.0000x reference)
import jax
import jax.numpy as jnp
from jax.experimental import pallas as pl
from jax.experimental.pallas import tpu as pltpu

N = 4096
R = 1024
T = N // R
NP1 = T * T
NP2 = T * (T + 1) // 2
_STARTS = [a * T - (a * (a - 1)) // 2 for a in range(T)]


def _proj_kernel(x_ref, a_ref, b_ref, c_ref, d_ref, u_ref, v_ref, w_ref):
    hi = jax.lax.Precision.HIGHEST
    a = a_ref[...]
    b = b_ref[...]
    c = c_ref[...]
    d = d_ref[...]
    ac = jnp.dot(a, c, precision=hi)
    ad = jnp.dot(a, d, precision=hi)
    bc = jnp.dot(b, c, precision=hi)
    bd = jnp.dot(b, d, precision=hi)
    xb = x_ref[...].astype(jnp.bfloat16)
    coef = jnp.concatenate([ac, bc + ad, bd], axis=1).astype(jnp.bfloat16)
    p = jnp.dot(xb, coef, preferred_element_type=jnp.float32)
    u_ref[...] = p[:, 0:1]
    v_ref[...] = p[:, 1:2]
    w_ref[...] = p[:, 2:3]


def _decode(g):
    is_p1 = g < NP1
    j1 = g // T
    b1 = g - j1 * T
    g2 = g - NP1
    a = jnp.int32(0)
    start_a = jnp.int32(0)
    for row in range(1, T):
        a = a + (g2 >= _STARTS[row]).astype(jnp.int32)
        start_a = jnp.where(g2 >= _STARTS[row], jnp.int32(_STARTS[row]), start_a)
    b2 = a + (g2 - start_a)
    j = jnp.where(is_p1, j1, a)
    b = jnp.where(is_p1, b1, b2)
    return is_p1, j, b


def _sweep_kernel(l_ref, w_ref, v_ref, u_ref, o_ref, s_scr, o_scr):
    g = pl.program_id(0)
    is_p1, j, b = _decode(g)

    tile = l_ref[...].astype(jnp.bfloat16)
    w_b = w_ref[pl.ds(b * R, R), :]
    vs_b = v_ref[pl.ds(b * R, R), :] + s_scr[pl.ds(b * R, R), :]
    use_vs = jnp.logical_or(jnp.logical_not(is_p1), b < j)
    vs_b = jnp.where(use_vs, vs_b, 0.0)
    rhs = jnp.concatenate([w_b, vs_b], axis=1).astype(jnp.bfloat16)
    p = jnp.dot(tile, rhs, preferred_element_type=jnp.float32)

    @pl.when(jnp.logical_and(is_p1, b == 0))
    def _init_row():
        s_scr[pl.ds(j * R, R), :] = p[:, 0:1]
        o_scr[pl.ds(j * R, R), :] = u_ref[pl.ds(j * R, R), :] + p[:, 1:2]

    @pl.when(jnp.logical_and(is_p1, b != 0))
    def _acc_row():
        s_scr[pl.ds(j * R, R), :] += p[:, 0:1]
        o_scr[pl.ds(j * R, R), :] += p[:, 1:2]

    @pl.when(jnp.logical_not(is_p1))
    def _acc_p2():
        o_scr[pl.ds(j * R, R), :] += p[:, 1:2]

    o_ref[...] = o_scr[pl.ds(j * R, R), :]


def _tile_index_map(g):
    _, j, b = _decode(g)
    return (j, b)


def _row_index_map(g):
    _, j, _ = _decode(g)
    return (j, 0)


def kernel(x, laplacian, W1, W2):
    a_m = W1[:, :, 0]
    b_m = W1[:, :, 1]
    c_m = W2[:, :, 0]
    d_m = W2[:, :, 1]
    vshape = jax.ShapeDtypeStruct((N, 1), jnp.float32)
    u_col, v_col, w_col = pl.pallas_call(
        _proj_kernel,
        out_shape=[vshape, vshape, vshape],
    )(x, a_m, b_m, c_m, d_m)

    vec_spec = pl.BlockSpec((N, 1), lambda g: (0, 0))
    out = pl.pallas_call(
        _sweep_kernel,
        grid=(NP1 + NP2,),
        in_specs=[pl.BlockSpec((R, R), _tile_index_map),
                  vec_spec, vec_spec, vec_spec],
        out_specs=pl.BlockSpec((R, 1), _row_index_map),
        out_shape=vshape,
        scratch_shapes=[pltpu.VMEM((N, 1), jnp.float32),
                        pltpu.VMEM((N, 1), jnp.float32)],
    )(laplacian, w_col, v_col, u_col)

    return out


# speedup vs baseline: 1.0510x; 1.0510x over previous
import jax
import jax.numpy as jnp
from jax.experimental import pallas as pl
from jax.experimental.pallas import tpu as pltpu

N = 4096
R = 512
T = N // R
R2 = 1024
T2 = N // R2
_STARTS = [a * T2 - (a * (a - 1)) // 2 for a in range(T2)]


def _proj_kernel(x_ref, a_ref, b_ref, c_ref, d_ref, u_ref, v_ref, w_ref):
    hi = jax.lax.Precision.HIGHEST
    a = a_ref[...]
    b = b_ref[...]
    c = c_ref[...]
    d = d_ref[...]
    ac = jnp.dot(a, c, precision=hi)
    ad = jnp.dot(a, d, precision=hi)
    bc = jnp.dot(b, c, precision=hi)
    bd = jnp.dot(b, d, precision=hi)
    xb = x_ref[...].astype(jnp.bfloat16)
    coef = jnp.concatenate([ac, bc + ad, bd], axis=1).astype(jnp.bfloat16)
    p = jnp.dot(xb, coef, preferred_element_type=jnp.float32)
    u_ref[...] = p[:, 0:1]
    v_ref[...] = p[:, 1:2]
    w_ref[...] = p[:, 2:3]


def _sweep1_kernel(l_ref, w_ref, v_ref, u_ref, s_ref, o_ref, s_scr):
    j = pl.program_id(0)
    blk = l_ref[...].astype(jnp.bfloat16)
    rows = jax.lax.broadcasted_iota(jnp.int32, (N, 1), 0)
    vs = jnp.where(rows < (j // 2) * R2, v_ref[...] + s_scr[...], 0.0)
    rhs = jnp.concatenate([w_ref[...], vs], axis=1).astype(jnp.bfloat16)
    p = jnp.dot(blk, rhs, preferred_element_type=jnp.float32)
    s_scr[pl.ds(j * R, R), :] = p[:, 0:1]
    s_ref[...] = p[:, 0:1]
    o_ref[...] = u_ref[...] + p[:, 1:2]


def _sweep2_kernel(l_ref, v_ref, s_ref, opart_ref, o_ref, acc_scr):
    g = pl.program_id(0)
    a = jnp.int32(0)
    start_a = jnp.int32(0)
    for row in range(1, T2):
        a = a + (g >= _STARTS[row]).astype(jnp.int32)
        start_a = jnp.where(g >= _STARTS[row], jnp.int32(_STARTS[row]), start_a)
    b = a + (g - start_a)

    tile = l_ref[...].astype(jnp.bfloat16)
    vs = (v_ref[pl.ds(b * R2, R2), :]
          + s_ref[pl.ds(b * R2, R2), :]).astype(jnp.bfloat16)
    prod = jnp.dot(tile, vs, preferred_element_type=jnp.float32)

    @pl.when(b == a)
    def _init():
        acc_scr[...] = opart_ref[...] + prod

    @pl.when(b != a)
    def _acc():
        acc_scr[...] += prod

    o_ref[...] = acc_scr[...]


def _tri_index_map(g):
    a = jnp.int32(0)
    start_a = jnp.int32(0)
    for row in range(1, T2):
        a = a + (g >= _STARTS[row]).astype(jnp.int32)
        start_a = jnp.where(g >= _STARTS[row], jnp.int32(_STARTS[row]), start_a)
    b = a + (g - start_a)
    return (a, b)


def kernel(x, laplacian, W1, W2):
    a_m = W1[:, :, 0]
    b_m = W1[:, :, 1]
    c_m = W2[:, :, 0]
    d_m = W2[:, :, 1]
    vshape = jax.ShapeDtypeStruct((N, 1), jnp.float32)
    u_col, v_col, w_col = pl.pallas_call(
        _proj_kernel,
        out_shape=[vshape, vshape, vshape],
    )(x, a_m, b_m, c_m, d_m)

    vec_spec = pl.BlockSpec((N, 1), lambda j: (0, 0))
    blk_col_spec = pl.BlockSpec((R, 1), lambda j: (j, 0))
    s_part, o_part = pl.pallas_call(
        _sweep1_kernel,
        grid=(T,),
        in_specs=[pl.BlockSpec((R, N), lambda j: (j, 0)),
                  vec_spec, vec_spec, blk_col_spec],
        out_specs=[blk_col_spec, blk_col_spec],
        out_shape=[vshape, vshape],
        scratch_shapes=[pltpu.VMEM((N, 1), jnp.float32)],
    )(laplacian, w_col, v_col, u_col)

    n_tri = T2 * (T2 + 1) // 2

    def _row_index_map(g):
        a, _ = _tri_index_map(g)
        return (a, 0)

    row_spec = pl.BlockSpec((R2, 1), _row_index_map)
    out = pl.pallas_call(
        _sweep2_kernel,
        grid=(n_tri,),
        in_specs=[pl.BlockSpec((R2, R2), _tri_index_map),
                  vec_spec, vec_spec, row_spec],
        out_specs=row_spec,
        out_shape=vshape,
        scratch_shapes=[pltpu.VMEM((R2, 1), jnp.float32)],
    )(laplacian, v_col, s_part, o_part)

    return out
